# Initial kernel scaffold; baseline (speedup 1.0000x reference)
import jax
import jax.numpy as jnp
from jax.experimental import pallas as pl


def kernel(fine_token_states, fine_token_mask):
    raise NotImplementedError("write your pallas kernel here")


# speedup vs baseline: 1.9070x; 1.9070x over previous
import functools

import jax
import jax.numpy as jnp
from jax.experimental import pallas as pl
from jax.experimental.pallas import tpu as pltpu

BLK = 64


def _body(x_ref, m_ref, diff_ref, mean_ref, cmask_ref, *, g):
    d = x_ref.shape[-1]
    x = x_ref[0]
    mc = m_ref[0]
    xm = x * mc
    xr = xm.reshape(g, BLK, d)
    s = jnp.sum(xr, axis=1)
    cnt = jnp.sum(mc.reshape(g, BLK, 1), axis=1)
    mean = s / (cnt + 1e-4)
    diff_ref[0] = (mean[:, None, :] - xr).reshape(g * BLK, d)
    mean_ref[0] = mean
    cmask_ref[0] = (cnt > 0).astype(x.dtype)


@functools.partial(jax.jit, static_argnames=("g",))
def _run(states, mask, g=8):
    b, f, d = states.shape
    nb = f // BLK
    mcol = mask.reshape(b, f, 1)
    grid = (b, nb // g)
    diff, mean, cmask = pl.pallas_call(
        functools.partial(_body, g=g),
        grid=grid,
        in_specs=[
            pl.BlockSpec((1, g * BLK, d), lambda i, j: (i, j, 0)),
            pl.BlockSpec((1, g * BLK, 1), lambda i, j: (i, j, 0)),
        ],
        out_specs=[
            pl.BlockSpec((1, g * BLK, d), lambda i, j: (i, j, 0)),
            pl.BlockSpec((1, g, d), lambda i, j: (i, j, 0)),
            pl.BlockSpec((1, g, 1), lambda i, j: (i, j, 0)),
        ],
        out_shape=[
            jax.ShapeDtypeStruct((b, f, d), states.dtype),
            jax.ShapeDtypeStruct((b, nb, d), states.dtype),
            jax.ShapeDtypeStruct((b, nb, 1), states.dtype),
        ],
        compiler_params=pltpu.CompilerParams(
            dimension_semantics=("parallel", "parallel"),
        ),
    )(states, mcol)
    return diff, mean, cmask


def kernel(fine_token_states, fine_token_mask):
    b, f, d = fine_token_states.shape
    nb = f // BLK
    diff, mean, cmask = _run(fine_token_states, fine_token_mask)
    indice = jnp.broadcast_to(jnp.arange(nb, dtype=jnp.int32)[None, :], (b, nb))
    return (mean, cmask.reshape(b, nb), diff.reshape(b, nb, BLK, d), indice)
